# baseline (speedup 1.0000x reference)
import functools

import jax
import jax.numpy as jnp
from jax import lax
from jax.experimental import pallas as pl
from jax.experimental.pallas import tpu as pltpu
from jax.experimental.pallas import tpu_sc as plsc

NN = 100000
NPAD = 100096
DF = 20
DG = 8
RPT = NPAD // 16
EE = 3200000
SL = 128
CPS = 16
NCHUNK = 49
EPAD = 32 * NCHUNK * CPS * SL
CH = CPS * SL
NBPAD = 12288
NIPAD = 90112
BR = 3128
F32 = jnp.float32


def _sc_mesh():
    return plsc.VectorSubcoreMesh(
        core_axis_name="c", subcore_axis_name="s", num_cores=2, num_subcores=16
    )


_SC_PARAMS = pltpu.CompilerParams(
    use_tc_tiling_on_sc=False, needs_layout_passes=False
)


def _sc_prep(col2d, bidx2d, iidx2d):
    nch = (EPAD // CH) // 2

    @functools.partial(
        pl.kernel,
        out_type=[
            jax.ShapeDtypeStruct((NPAD,), F32),
            jax.ShapeDtypeStruct((NPAD,), F32),
            jax.ShapeDtypeStruct((NPAD,), F32),
        ],
        mesh=_sc_mesh(),
        scratch_types=[
            pltpu.VMEM((2, CH), jnp.int32),
            pltpu.VMEM((RPT,), F32),
            pltpu.VMEM((RPT,), F32),
            [pltpu.SemaphoreType.DMA] * 2,
        ],
        compiler_params=_SC_PARAMS,
    )
    def k(colr, bidxr, iidxr, deg0, deg1, flago, cbuf, dbuf, fbuf, isem):
        c = lax.axis_index("c")
        s = lax.axis_index("s")
        base = s * RPT
        ones = jnp.full((16,), 1.0, F32)
        twos = jnp.full((16,), 2.0, F32)

        def zfill(i, _):
            idx = pl.ds(i * 16, 16)
            dbuf[idx] = jnp.zeros((16,), F32)
            fbuf[idx] = jnp.zeros((16,), F32)
            return _

        lax.fori_loop(0, RPT // 16, zfill, None)

        def scan_buf(b, vals, add):
            def vec(j, _):
                for kk in range(SL // 16):
                    idx16 = cbuf[b, pl.ds(j * SL + kk * 16, 16)]
                    local = idx16 - base
                    mask = (local >= 0) & (local < RPT)
                    safe = jnp.where(mask, local, 0)
                    if add:
                        plsc.addupdate_scatter(dbuf, [safe], ones, mask=mask)
                    else:
                        plsc.store_scatter(fbuf, [safe], vals, mask=mask)
                return _

            lax.fori_loop(0, CPS, vec, None)

        def scan_many(ref, first, count, vals, add):
            assert count % 2 == 0
            for b in range(2):
                pltpu.async_copy(ref.at[first + b], cbuf.at[b], isem[b])

            def pair(i2, _):
                for b in range(2):
                    i = i2 * 2 + b
                    pltpu.make_async_copy(
                        ref.at[first], cbuf.at[b], isem[b]).wait()
                    scan_buf(b, vals, add)

                    @pl.when(i + 2 < count)
                    def _():
                        pltpu.async_copy(
                            ref.at[first + i + 2], cbuf.at[b], isem[b])
                return _

            lax.fori_loop(0, count // 2, pair, None)

        scan_many(colr, c * nch, nch, None, True)

        @pl.when(c == 0)
        def _():
            scan_many(bidxr, 0, NBPAD // CH, ones, False)
            scan_many(iidxr, 0, NIPAD // CH, twos, False)

            pltpu.sync_copy(fbuf, flago.at[pl.ds(base, RPT)])
            pltpu.sync_copy(dbuf, deg0.at[pl.ds(base, RPT)])

        @pl.when(c == 1)
        def _():
            pltpu.sync_copy(dbuf, deg1.at[pl.ds(base, RPT)])

    return k(col2d, bidx2d, iidx2d)


def _sc_layer(ga, gb, gc, row2d, col2d):
    @functools.partial(
        pl.kernel,
        out_type=[jax.ShapeDtypeStruct((NPAD, DG), F32)] * 6,
        mesh=_sc_mesh(),
        scratch_types=[
            pltpu.VMEM((2, CPS * SL), jnp.int32),
            pltpu.VMEM((2, CPS * SL), jnp.int32),
            pltpu.VMEM((2, CPS * SL, DG), F32),
            pltpu.VMEM((RPT // 2, DG), F32),
            pltpu.VMEM_SHARED((NPAD, DG), F32),
            [pltpu.SemaphoreType.DMA] * 2,
            [pltpu.SemaphoreType.DMA] * 2,
        ],
        compiler_params=_SC_PARAMS,
    )
    def k(gha, ghb, ghc, rowr, colr, oa0, oa1, ob0, ob1, oc0, oc1,
          rowv, colv, vals, bnc, acc, gsem, ssem):
        c = lax.axis_index("c")
        s = lax.axis_index("s")
        wid = c * 16 + s
        qr = RPT // 2

        def init_acc(gh):
            for q in range(2):
                qrows = pl.ds(s * RPT + q * qr, qr)
                pltpu.sync_copy(gh.at[qrows], bnc)
                pltpu.sync_copy(bnc, acc.at[qrows])

        def load_and_fire(gh, i, b):
            ch = wid * NCHUNK + i
            pltpu.sync_copy(rowr.at[ch], rowv.at[b])
            pltpu.sync_copy(colr.at[ch], colv.at[b])
            pltpu.async_copy(gh.at[rowv.at[b]], vals.at[b], gsem[b])

        def drain(gh, sem, b):
            pltpu.make_async_copy(
                gh.at[pl.ds(0, CPS * SL)], vals.at[b], sem).wait()

        def edge_pass(gh):
            for b in range(2):
                load_and_fire(gh, b, b)

            def chunk2(i2, _):
                for b in range(2):
                    i = i2 * 2 + b

                    @pl.when(i < NCHUNK)
                    def _():
                        drain(gh, gsem[b], b)
                        pltpu.async_copy(
                            vals.at[b], acc.at[colv.at[b]], ssem[b], add=True)
                        nxt = i + 2

                        @pl.when(nxt < NCHUNK)
                        def _():
                            drain(gh, ssem[b], b)
                            load_and_fire(gh, nxt, b)
                return _

            lax.fori_loop(0, (NCHUNK + 1) // 2, chunk2, None)
            for b in range(2):
                drain(gh, ssem[b], b)

        def write_out(o0, o1):
            def out_core(oref):
                for q in range(2):
                    qrows = pl.ds(s * RPT + q * qr, qr)
                    pltpu.sync_copy(acc.at[qrows], bnc)
                    pltpu.sync_copy(bnc, oref.at[qrows])

            @pl.when(c == 0)
            def _():
                out_core(o0)

            @pl.when(c == 1)
            def _():
                out_core(o1)

        for gh, o0, o1 in ((gha, oa0, oa1), (ghb, ob0, ob1), (ghc, oc0, oc1)):
            init_acc(gh)
            plsc.subcore_barrier()
            edge_pass(gh)
            plsc.subcore_barrier()
            write_out(o0, o1)

    return k(ga, gb, gc, row2d, col2d)


def _full(shape):
    return pl.BlockSpec(shape, lambda i: tuple(0 for _ in shape))


def _rows(width):
    return pl.BlockSpec((BR, width), lambda i: (i, 0))


def _split_g(g):
    pad = jnp.zeros((g.shape[0], 3 * DG - DF), F32)
    return g[:, 0:DG], g[:, DG:2 * DG], jnp.concatenate([g[:, 2 * DG:], pad], 1)


def _tc_encoder(xp, yp, flagf, deg0, deg1, Wb1, bb1, Wb2, bb2, Wi1, bi1, Wi2,
                bi2, Wc1):
    def body(xr, yr, fr, d0r, d1r, wb1, rb1, wb2, rb2, wi1, ri1, wi2, ri2, wc1,
             gao, gbo, gco, dio):
        xv = xr[...]
        yv = yr[...]
        fv = fr[...]
        di = lax.rsqrt(d0r[...] + d1r[...] + 1.0)
        tb = jnp.dot(xv, wb1[0:2, :], preferred_element_type=F32)
        tb = tb + yv * wb1[2:3, :] + rb1[...]
        tb = jnp.maximum(tb, 0.0)
        tb = jnp.dot(tb, wb2[...], preferred_element_type=F32) + rb2[...]
        ti = jnp.dot(xv, wi1[...], preferred_element_type=F32) + ri1[...]
        ti = jnp.maximum(ti, 0.0)
        ti = jnp.dot(ti, wi2[...], preferred_element_type=F32) + ri2[...]
        h = jnp.where(fv == 2.0, ti, jnp.where(fv == 1.0, tb, 0.0))
        h = jnp.maximum(h, 0.0)
        g = jnp.dot(h, wc1[...], preferred_element_type=F32) * di
        gao[...], gbo[...], gco[...] = _split_g(g)
        dio[...] = di

    return pl.pallas_call(
        body,
        grid=(NPAD // BR,),
        in_specs=[
            _rows(2), _rows(1), _rows(1), _rows(1), _rows(1),
            _full((3, DF)), _full((1, DF)), _full((DF, DF)), _full((1, DF)),
            _full((2, DF)), _full((1, DF)), _full((DF, DF)), _full((1, DF)),
            _full((DF, DF)),
        ],
        out_specs=[_rows(DG), _rows(DG), _rows(DG), _rows(1)],
        out_shape=[
            jax.ShapeDtypeStruct((NPAD, DG), F32),
            jax.ShapeDtypeStruct((NPAD, DG), F32),
            jax.ShapeDtypeStruct((NPAD, DG), F32),
            jax.ShapeDtypeStruct((NPAD, 1), F32),
        ],
    )(xp, yp, flagf, deg0, deg1, Wb1, bb1, Wb2, bb2, Wi1, bi1, Wi2, bi2, Wc1)


def _combine(parts, g3, di, bias):
    segs = [parts[2 * i][...] + parts[2 * i + 1][...] - g3[i][...]
            for i in range(3)]
    acc = jnp.concatenate(segs, axis=1)[:, :DF]
    return jnp.maximum(di * acc + bias[...], 0.0)


def _tc_layer(aa0, aa1, ab0, ab1, ac0, ac1, ga, gb, gc, dinv, bl, Wn):
    def body(a0r, a1r, b0r, b1r, c0r, c1r, gar, gbr, gcr, dir_, blr, wnr,
             gao, gbo, gco):
        di = dir_[...]
        h = _combine([a0r, a1r, b0r, b1r, c0r, c1r], [gar, gbr, gcr], di, blr)
        g = jnp.dot(h, wnr[...], preferred_element_type=F32) * di
        gao[...], gbo[...], gco[...] = _split_g(g)

    return pl.pallas_call(
        body,
        grid=(NPAD // BR,),
        in_specs=[_rows(DG)] * 9 + [_rows(1), _full((1, DF)), _full((DF, DF))],
        out_specs=[_rows(DG), _rows(DG), _rows(DG)],
        out_shape=[
            jax.ShapeDtypeStruct((NPAD, DG), F32),
            jax.ShapeDtypeStruct((NPAD, DG), F32),
            jax.ShapeDtypeStruct((NPAD, DG), F32),
        ],
    )(aa0, aa1, ab0, ab1, ac0, ac1, ga, gb, gc, dinv, bl, Wn)


def _tc_final(aa0, aa1, ab0, ab1, ac0, ac1, ga, gb, gc, dinv, bc6, Wf1, bf1,
              Wf2, bf2):
    def body(a0r, a1r, b0r, b1r, c0r, c1r, gar, gbr, gcr, dir_, b6r, w1r, b1r_,
             w2r, b2r, oo):
        di = dir_[...]
        h = _combine([a0r, a1r, b0r, b1r, c0r, c1r], [gar, gbr, gcr], di, b6r)
        t = jnp.dot(h, w1r[...], preferred_element_type=F32) + b1r_[...]
        t = jnp.maximum(t, 0.0)
        oo[...] = jnp.dot(t, w2r[...], preferred_element_type=F32) + b2r[...]

    return pl.pallas_call(
        body,
        grid=(NPAD // BR,),
        in_specs=[_rows(DG)] * 9 + [
            _rows(1), _full((1, DF)), _full((DF, DF)), _full((1, DF)),
            _full((DF, 1)), _full((1, 1)),
        ],
        out_specs=_rows(1),
        out_shape=jax.ShapeDtypeStruct((NPAD, 1), F32),
    )(aa0, aa1, ab0, ab1, ac0, ac1, ga, gb, gc, dinv, bc6, Wf1, bf1, Wf2, bf2)


def kernel(x, y, edge_index, boundary_index, interior_index,
           Wb1, bb1, Wb2, bb2, Wi1, bi1, Wi2, bi2,
           Wc1, bc1, Wc2, bc2, Wc3, bc3, Wc4, bc4, Wc5, bc5, Wc6, bc6,
           Wf1, bf1, Wf2, bf2):
    i32 = jnp.int32
    row = edge_index[0]
    col = edge_index[1]
    row2d = jnp.concatenate(
        [row, jnp.zeros((EPAD - EE,), i32)]).reshape(EPAD // CH, CH)
    col2d = jnp.concatenate(
        [col, jnp.full((EPAD - EE,), NN, i32)]).reshape(EPAD // CH, CH)
    bidx2d = jnp.concatenate(
        [boundary_index, jnp.full((NBPAD - boundary_index.shape[0],), NN, i32)]
    ).reshape(NBPAD // CH, CH)
    iidx2d = jnp.concatenate(
        [interior_index, jnp.full((NIPAD - interior_index.shape[0],), NN, i32)]
    ).reshape(NIPAD // CH, CH)

    xp = jnp.zeros((NPAD, 2), F32).at[:NN].set(x)
    yp = jnp.zeros((NPAD, 1), F32).at[:NN, 0].set(y)

    deg0, deg1, flagf = _sc_prep(col2d, bidx2d, iidx2d)
    deg0 = deg0.reshape(NPAD, 1)
    deg1 = deg1.reshape(NPAD, 1)
    flagf = flagf.reshape(NPAD, 1)

    br = lambda b: b.reshape(1, -1)
    ga, gb, gc, dinv = _tc_encoder(xp, yp, flagf, deg0, deg1,
                                   Wb1, br(bb1), Wb2, br(bb2),
                                   Wi1, br(bi1), Wi2, br(bi2), Wc1)

    for bl, Wn in ((bc1, Wc2), (bc2, Wc3), (bc3, Wc4), (bc4, Wc5), (bc5, Wc6)):
        parts = _sc_layer(ga, gb, gc, row2d, col2d)
        ga, gb, gc = _tc_layer(*parts, ga, gb, gc, dinv, br(bl), Wn)

    parts = _sc_layer(ga, gb, gc, row2d, col2d)
    out = _tc_final(*parts, ga, gb, gc, dinv, br(bc6), Wf1, br(bf1), Wf2,
                    bf2.reshape(1, 1))
    return out[:NN]


# speedup vs baseline: 1.1259x; 1.1259x over previous
import functools

import jax
import jax.numpy as jnp
from jax import lax
from jax.experimental import pallas as pl
from jax.experimental.pallas import tpu as pltpu
from jax.experimental.pallas import tpu_sc as plsc

NN = 100000
NPAD = 100096
DF = 20
DG = 8
RPT = NPAD // 16
EE = 3200000
SL = 128
CPS = 16
NCHUNK = 49
EPAD = 32 * NCHUNK * CPS * SL
CH = CPS * SL
NBPAD = 12288
NIPAD = 90112
BR = 3128
F32 = jnp.float32


def _sc_mesh():
    return plsc.VectorSubcoreMesh(
        core_axis_name="c", subcore_axis_name="s", num_cores=2, num_subcores=16
    )


_SC_PARAMS = pltpu.CompilerParams(
    use_tc_tiling_on_sc=False, needs_layout_passes=False
)


def _sc_prep(col2d, bidx2d, iidx2d):
    nch = (EPAD // CH) // 2

    @functools.partial(
        pl.kernel,
        out_type=[
            jax.ShapeDtypeStruct((NPAD,), F32),
            jax.ShapeDtypeStruct((NPAD,), F32),
            jax.ShapeDtypeStruct((NPAD,), F32),
        ],
        mesh=_sc_mesh(),
        scratch_types=[
            pltpu.VMEM((2, CH), jnp.int32),
            pltpu.VMEM((RPT,), F32),
            pltpu.VMEM((RPT,), F32),
            [pltpu.SemaphoreType.DMA] * 2,
        ],
        compiler_params=_SC_PARAMS,
    )
    def k(colr, bidxr, iidxr, deg0, deg1, flago, cbuf, dbuf, fbuf, isem):
        c = lax.axis_index("c")
        s = lax.axis_index("s")
        base = s * RPT
        ones = jnp.full((16,), 1.0, F32)
        twos = jnp.full((16,), 2.0, F32)

        def zfill(i, _):
            idx = pl.ds(i * 16, 16)
            dbuf[idx] = jnp.zeros((16,), F32)
            fbuf[idx] = jnp.zeros((16,), F32)
            return _

        lax.fori_loop(0, RPT // 16, zfill, None)

        def scan_buf(b, vals, add):
            def vec(j, _):
                for kk in range(SL // 16):
                    idx16 = cbuf[b, pl.ds(j * SL + kk * 16, 16)]
                    local = idx16 - base
                    mask = (local >= 0) & (local < RPT)
                    safe = jnp.where(mask, local, 0)
                    if add:
                        plsc.addupdate_scatter(dbuf, [safe], ones, mask=mask)
                    else:
                        plsc.store_scatter(fbuf, [safe], vals, mask=mask)
                return _

            lax.fori_loop(0, CPS, vec, None)

        def scan_many(ref, first, count, vals, add):
            assert count % 2 == 0
            for b in range(2):
                pltpu.async_copy(ref.at[first + b], cbuf.at[b], isem[b])

            def pair(i2, _):
                for b in range(2):
                    i = i2 * 2 + b
                    pltpu.make_async_copy(
                        ref.at[first], cbuf.at[b], isem[b]).wait()
                    scan_buf(b, vals, add)

                    @pl.when(i + 2 < count)
                    def _():
                        pltpu.async_copy(
                            ref.at[first + i + 2], cbuf.at[b], isem[b])
                return _

            lax.fori_loop(0, count // 2, pair, None)

        scan_many(colr, c * nch, nch, None, True)

        @pl.when(c == 0)
        def _():
            scan_many(bidxr, 0, NBPAD // CH, ones, False)
            scan_many(iidxr, 0, NIPAD // CH, twos, False)

            pltpu.sync_copy(fbuf, flago.at[pl.ds(base, RPT)])
            pltpu.sync_copy(dbuf, deg0.at[pl.ds(base, RPT)])

        @pl.when(c == 1)
        def _():
            pltpu.sync_copy(dbuf, deg1.at[pl.ds(base, RPT)])

    return k(col2d, bidx2d, iidx2d)


def _sc_scatter(gp, row2d, col2d):
    @functools.partial(
        pl.kernel,
        out_type=[jax.ShapeDtypeStruct((NPAD, DG), F32)] * 2,
        mesh=_sc_mesh(),
        scratch_types=[
            pltpu.VMEM((2, CPS * SL), jnp.int32),
            pltpu.VMEM((2, CPS * SL), jnp.int32),
            pltpu.VMEM((2, CPS * SL, DG), F32),
            pltpu.VMEM((RPT // 2, DG), F32),
            pltpu.VMEM_SHARED((NPAD, DG), F32),
            [pltpu.SemaphoreType.DMA] * 2,
            [pltpu.SemaphoreType.DMA] * 2,
        ],
        compiler_params=_SC_PARAMS,
    )
    def k(gha, rowr, colr, oa0, oa1,
          rowv, colv, vals, bnc, acc, gsem, ssem):
        c = lax.axis_index("c")
        s = lax.axis_index("s")
        wid = c * 16 + s
        qr = RPT // 2

        def init_acc(gh):
            for q in range(2):
                qrows = pl.ds(s * RPT + q * qr, qr)
                pltpu.sync_copy(gh.at[qrows], bnc)
                pltpu.sync_copy(bnc, acc.at[qrows])

        def load_and_fire(gh, i, b):
            ch = wid * NCHUNK + i
            pltpu.sync_copy(rowr.at[ch], rowv.at[b])
            pltpu.sync_copy(colr.at[ch], colv.at[b])
            pltpu.async_copy(gh.at[rowv.at[b]], vals.at[b], gsem[b])

        def drain(gh, sem, b):
            pltpu.make_async_copy(
                gh.at[pl.ds(0, CPS * SL)], vals.at[b], sem).wait()

        def edge_pass(gh):
            for b in range(2):
                load_and_fire(gh, b, b)

            def chunk2(i2, _):
                for b in range(2):
                    i = i2 * 2 + b

                    @pl.when(i < NCHUNK)
                    def _():
                        drain(gh, gsem[b], b)
                        pltpu.async_copy(
                            vals.at[b], acc.at[colv.at[b]], ssem[b], add=True)
                        nxt = i + 2

                        @pl.when(nxt < NCHUNK)
                        def _():
                            drain(gh, ssem[b], b)
                            load_and_fire(gh, nxt, b)
                return _

            lax.fori_loop(0, (NCHUNK + 1) // 2, chunk2, None)
            for b in range(2):
                drain(gh, ssem[b], b)

        def write_out(o0, o1):
            def out_core(oref):
                for q in range(2):
                    qrows = pl.ds(s * RPT + q * qr, qr)
                    pltpu.sync_copy(acc.at[qrows], bnc)
                    pltpu.sync_copy(bnc, oref.at[qrows])

            @pl.when(c == 0)
            def _():
                out_core(o0)

            @pl.when(c == 1)
            def _():
                out_core(o1)

        init_acc(gha)
        plsc.subcore_barrier()
        edge_pass(gha)
        plsc.subcore_barrier()
        write_out(oa0, oa1)

    return k(gp, row2d, col2d)


def _full(shape):
    return pl.BlockSpec(shape, lambda i: tuple(0 for _ in shape))


def _rows(width):
    return pl.BlockSpec((BR, width), lambda i: (i, 0))


def _split_g(g):
    pad = jnp.zeros((g.shape[0], 3 * DG - DF), F32)
    return g[:, 0:DG], g[:, DG:2 * DG], jnp.concatenate([g[:, 2 * DG:], pad], 1)


def _tc_encoder(xp, yp, flagf, deg0, deg1, Wb1, bb1, Wb2, bb2, Wi1, bi1, Wi2,
                bi2, Wc1):
    def body(xr, yr, fr, d0r, d1r, wb1, rb1, wb2, rb2, wi1, ri1, wi2, ri2, wc1,
             gao, gbo, gco, dio):
        xv = xr[...]
        yv = yr[...]
        fv = fr[...]
        di = lax.rsqrt(d0r[...] + d1r[...] + 1.0)
        tb = jnp.dot(xv, wb1[0:2, :], preferred_element_type=F32)
        tb = tb + yv * wb1[2:3, :] + rb1[...]
        tb = jnp.maximum(tb, 0.0)
        tb = jnp.dot(tb, wb2[...], preferred_element_type=F32) + rb2[...]
        ti = jnp.dot(xv, wi1[...], preferred_element_type=F32) + ri1[...]
        ti = jnp.maximum(ti, 0.0)
        ti = jnp.dot(ti, wi2[...], preferred_element_type=F32) + ri2[...]
        h = jnp.where(fv == 2.0, ti, jnp.where(fv == 1.0, tb, 0.0))
        h = jnp.maximum(h, 0.0)
        g = jnp.dot(h, wc1[...], preferred_element_type=F32) * di
        gao[...], gbo[...], gco[...] = _split_g(g)
        dio[...] = di

    return pl.pallas_call(
        body,
        grid=(NPAD // BR,),
        in_specs=[
            _rows(2), _rows(1), _rows(1), _rows(1), _rows(1),
            _full((3, DF)), _full((1, DF)), _full((DF, DF)), _full((1, DF)),
            _full((2, DF)), _full((1, DF)), _full((DF, DF)), _full((1, DF)),
            _full((DF, DF)),
        ],
        out_specs=[_rows(DG), _rows(DG), _rows(DG), _rows(1)],
        out_shape=[
            jax.ShapeDtypeStruct((NPAD, DG), F32),
            jax.ShapeDtypeStruct((NPAD, DG), F32),
            jax.ShapeDtypeStruct((NPAD, DG), F32),
            jax.ShapeDtypeStruct((NPAD, 1), F32),
        ],
    )(xp, yp, flagf, deg0, deg1, Wb1, bb1, Wb2, bb2, Wi1, bi1, Wi2, bi2, Wc1)


def _combine(parts, g3, di, bias):
    segs = [parts[2 * i][...] + parts[2 * i + 1][...] - g3[i][...]
            for i in range(3)]
    acc = jnp.concatenate(segs, axis=1)[:, :DF]
    return jnp.maximum(di * acc + bias[...], 0.0)


def _tc_layer(aa0, aa1, ab0, ab1, ac0, ac1, ga, gb, gc, dinv, bl, Wn):
    def body(a0r, a1r, b0r, b1r, c0r, c1r, gar, gbr, gcr, dir_, blr, wnr,
             gao, gbo, gco):
        di = dir_[...]
        h = _combine([a0r, a1r, b0r, b1r, c0r, c1r], [gar, gbr, gcr], di, blr)
        g = jnp.dot(h, wnr[...], preferred_element_type=F32) * di
        gao[...], gbo[...], gco[...] = _split_g(g)

    return pl.pallas_call(
        body,
        grid=(NPAD // BR,),
        in_specs=[_rows(DG)] * 9 + [_rows(1), _full((1, DF)), _full((DF, DF))],
        out_specs=[_rows(DG), _rows(DG), _rows(DG)],
        out_shape=[
            jax.ShapeDtypeStruct((NPAD, DG), F32),
            jax.ShapeDtypeStruct((NPAD, DG), F32),
            jax.ShapeDtypeStruct((NPAD, DG), F32),
        ],
    )(aa0, aa1, ab0, ab1, ac0, ac1, ga, gb, gc, dinv, bl, Wn)


def _tc_final(aa0, aa1, ab0, ab1, ac0, ac1, ga, gb, gc, dinv, bc6, Wf1, bf1,
              Wf2, bf2):
    def body(a0r, a1r, b0r, b1r, c0r, c1r, gar, gbr, gcr, dir_, b6r, w1r, b1r_,
             w2r, b2r, oo):
        di = dir_[...]
        h = _combine([a0r, a1r, b0r, b1r, c0r, c1r], [gar, gbr, gcr], di, b6r)
        t = jnp.dot(h, w1r[...], preferred_element_type=F32) + b1r_[...]
        t = jnp.maximum(t, 0.0)
        oo[...] = jnp.dot(t, w2r[...], preferred_element_type=F32) + b2r[...]

    return pl.pallas_call(
        body,
        grid=(NPAD // BR,),
        in_specs=[_rows(DG)] * 9 + [
            _rows(1), _full((1, DF)), _full((DF, DF)), _full((1, DF)),
            _full((DF, 1)), _full((1, 1)),
        ],
        out_specs=_rows(1),
        out_shape=jax.ShapeDtypeStruct((NPAD, 1), F32),
    )(aa0, aa1, ab0, ab1, ac0, ac1, ga, gb, gc, dinv, bc6, Wf1, bf1, Wf2, bf2)


def kernel(x, y, edge_index, boundary_index, interior_index,
           Wb1, bb1, Wb2, bb2, Wi1, bi1, Wi2, bi2,
           Wc1, bc1, Wc2, bc2, Wc3, bc3, Wc4, bc4, Wc5, bc5, Wc6, bc6,
           Wf1, bf1, Wf2, bf2):
    i32 = jnp.int32
    row = edge_index[0]
    col = edge_index[1]
    row2d = jnp.concatenate(
        [row, jnp.zeros((EPAD - EE,), i32)]).reshape(EPAD // CH, CH)
    col2d = jnp.concatenate(
        [col, jnp.full((EPAD - EE,), NN, i32)]).reshape(EPAD // CH, CH)
    bidx2d = jnp.concatenate(
        [boundary_index, jnp.full((NBPAD - boundary_index.shape[0],), NN, i32)]
    ).reshape(NBPAD // CH, CH)
    iidx2d = jnp.concatenate(
        [interior_index, jnp.full((NIPAD - interior_index.shape[0],), NN, i32)]
    ).reshape(NIPAD // CH, CH)

    xp = jnp.zeros((NPAD, 2), F32).at[:NN].set(x)
    yp = jnp.zeros((NPAD, 1), F32).at[:NN, 0].set(y)

    deg0, deg1, flagf = _sc_prep(col2d, bidx2d, iidx2d)
    deg0 = deg0.reshape(NPAD, 1)
    deg1 = deg1.reshape(NPAD, 1)
    flagf = flagf.reshape(NPAD, 1)

    br = lambda b: b.reshape(1, -1)
    ga, gb, gc, dinv = _tc_encoder(xp, yp, flagf, deg0, deg1,
                                   Wb1, br(bb1), Wb2, br(bb2),
                                   Wi1, br(bi1), Wi2, br(bi2), Wc1)

    def sc_layer(ga, gb, gc):
        return (*_sc_scatter(ga, row2d, col2d),
                *_sc_scatter(gb, row2d, col2d),
                *_sc_scatter(gc, row2d, col2d))

    for bl, Wn in ((bc1, Wc2), (bc2, Wc3), (bc3, Wc4), (bc4, Wc5), (bc5, Wc6)):
        parts = sc_layer(ga, gb, gc)
        ga, gb, gc = _tc_layer(*parts, ga, gb, gc, dinv, br(bl), Wn)

    parts = sc_layer(ga, gb, gc)
    out = _tc_final(*parts, ga, gb, gc, dinv, br(bc6), Wf1, br(bf1), Wf2,
                    bf2.reshape(1, 1))
    return out[:NN]
